# Initial kernel scaffold; baseline (speedup 1.0000x reference)
import jax
import jax.numpy as jnp
from jax.experimental import pallas as pl


def kernel(node_attr, edge_index, edge_attr, edge_sh, W1, b1, W2, b2):
    raise NotImplementedError("write your pallas kernel here")


# speedup vs baseline: 125.8947x; 125.8947x over previous
import functools

import numpy as np
import jax
import jax.numpy as jnp
from jax import lax
from jax.experimental import pallas as pl
from jax.experimental.pallas import tpu as pltpu
from jax.experimental.pallas import tpu_sc as plsc

_N = 10000
_E = 160000
_EP = 163840
_NP = 10400
_CH = 128
_NW = 32
_PERW = _EP // _NW
_NCHUNK = _PERW // _CH
_CNTW = 16
_TE = 256
_TN = 400

_A0 = float(np.sqrt(1.0 / 32.0))
_A110 = float(np.sqrt(1.0 / 96.0))


def _build_consts():
    e1 = np.zeros((64, 48), np.float32)
    for u in range(16):
        for k in range(3):
            e1[16 + 3 * u + k, 16 * k + u] = 1.0
    r1 = np.zeros((16, 256), np.float32)
    for u in range(16):
        r1[u, u * 16:(u + 1) * 16] = 1.0
    s1 = np.zeros((256, 16), np.float32)
    for j in range(256):
        s1[j, j % 16] = 1.0
    g = np.zeros((48, 48), np.float32)
    for k in range(3):
        for w in range(16):
            g[16 * k + w, 3 * w + k] = 1.0
    return e1, r1, s1, g


_E1, _R1, _S1, _G = _build_consts()


def _dense_body(ea, xs, sh, w1, b1, w2, b2, e1, r1, s1, g, out):
    h = jnp.dot(ea[...], w1[...], preferred_element_type=jnp.float32) + b1[...]
    h = h * jax.nn.sigmoid(h)
    w = jnp.dot(h, w2[...], preferred_element_type=jnp.float32) + b2[...]
    x0 = xs[:, 0:16]
    x1c = jnp.dot(xs[...], e1[...], preferred_element_type=jnp.float32)
    sh0 = sh[:, 0:1]
    v000 = x0 * (sh0 * _A0)
    v110 = (x1c[:, 0:16] * sh[:, 1:2]
            + x1c[:, 16:32] * sh[:, 2:3]
            + x1c[:, 32:48] * sh[:, 3:4]) * _A110
    r = r1[...]
    s = s1[...]
    p0 = (jnp.dot(v000, r, preferred_element_type=jnp.float32) * w[:, 0:256]
          + jnp.dot(v110, r, preferred_element_type=jnp.float32) * w[:, 768:1024])
    out0 = jnp.dot(p0, s, preferred_element_type=jnp.float32)
    t011 = jnp.dot(jnp.dot(x0, r, preferred_element_type=jnp.float32) * w[:, 256:512],
                   s, preferred_element_type=jnp.float32)
    w101 = w[:, 512:768]
    o1 = []
    for k in range(3):
        vk = x1c[:, 16 * k:16 * (k + 1)] * (sh0 * _A0)
        o1k = (jnp.dot(jnp.dot(vk, r, preferred_element_type=jnp.float32) * w101,
                       s, preferred_element_type=jnp.float32)
               + t011 * (sh[:, 1 + k:2 + k] * _A0))
        o1.append(o1k)
    o1cat = jnp.dot(jnp.concatenate(o1, axis=1), g[...],
                    preferred_element_type=jnp.float32)
    out[...] = jnp.concatenate([out0, o1cat], axis=1)


def _dense(ea, xs, sh, w1, b1, w2, b2):
    grid = _EP // _TE
    return pl.pallas_call(
        _dense_body,
        grid=(grid,),
        in_specs=[
            pl.BlockSpec((_TE, 16), lambda i: (i, 0)),
            pl.BlockSpec((_TE, 64), lambda i: (i, 0)),
            pl.BlockSpec((_TE, 4), lambda i: (i, 0)),
            pl.BlockSpec((16, 16), lambda i: (0, 0)),
            pl.BlockSpec((1, 16), lambda i: (0, 0)),
            pl.BlockSpec((16, 1024), lambda i: (0, 0)),
            pl.BlockSpec((1, 1024), lambda i: (0, 0)),
            pl.BlockSpec((64, 48), lambda i: (0, 0)),
            pl.BlockSpec((16, 256), lambda i: (0, 0)),
            pl.BlockSpec((256, 16), lambda i: (0, 0)),
            pl.BlockSpec((48, 48), lambda i: (0, 0)),
        ],
        out_specs=pl.BlockSpec((_TE, 64), lambda i: (i, 0)),
        out_shape=jax.ShapeDtypeStruct((_EP, 64), jnp.float32),
        compiler_params=pltpu.CompilerParams(
            dimension_semantics=("parallel",)),
    )(ea, xs, sh, w1, b1, w2, b2,
      jnp.asarray(_E1), jnp.asarray(_R1), jnp.asarray(_S1), jnp.asarray(_G))


def _gather(nodes, src_p):
    mesh = plsc.VectorSubcoreMesh(core_axis_name="c", subcore_axis_name="s")

    @functools.partial(
        pl.kernel, mesh=mesh,
        out_type=jax.ShapeDtypeStruct((_EP, 64), jnp.float32),
        scratch_types=[
            pltpu.VMEM((_CH,), jnp.int32),
            pltpu.VMEM((_CH, 64), jnp.float32),
            pltpu.SemaphoreType.DMA,
        ],
        compiler_params=pltpu.CompilerParams(use_tc_tiling_on_sc=False),
    )
    def gk(nodes_hbm, src_hbm, out_hbm, idx_v, rows_v, sem):
        wid = lax.axis_index("s") * 2 + lax.axis_index("c")

        def body(i, carry):
            base = wid * np.int32(_PERW) + i * np.int32(_CH)
            pltpu.sync_copy(src_hbm.at[pl.ds(base, _CH)], idx_v)
            pltpu.async_copy(nodes_hbm.at[idx_v], rows_v, sem).wait()
            pltpu.sync_copy(rows_v, out_hbm.at[pl.ds(base, _CH)])
            return carry

        lax.fori_loop(np.int32(0), np.int32(_NCHUNK), body, np.int32(0))

    return gk(nodes, src_p)


def _scatter(msgs, dst_p, zsum, zcnt, ones):
    mesh = plsc.VectorSubcoreMesh(core_axis_name="c", subcore_axis_name="s")

    @functools.partial(
        pl.kernel, mesh=mesh,
        out_type=(jax.ShapeDtypeStruct((2 * _NP, 64), jnp.float32),
                  jax.ShapeDtypeStruct((2 * _NP, _CNTW), jnp.float32)),
        scratch_types=[
            pltpu.VMEM((_CH,), jnp.int32),
            pltpu.VMEM((_CH, 64), jnp.float32),
            pltpu.VMEM((_CH, _CNTW), jnp.float32),
            pltpu.VMEM_SHARED((_NP, 64), jnp.float32),
            pltpu.VMEM_SHARED((_NP, _CNTW), jnp.float32),
        ],
        compiler_params=pltpu.CompilerParams(use_tc_tiling_on_sc=False),
    )
    def sk(msgs_hbm, dst_hbm, zs_hbm, zc_hbm, on_hbm,
           sums_out, cnts_out, idx_v, rows_v, ones_v, ssum, scnt):
        c = lax.axis_index("c")
        s = lax.axis_index("s")
        wid = s * 2 + c
        rz = _NP // 16
        srz = s * np.int32(rz)
        pltpu.sync_copy(zs_hbm.at[pl.ds(srz, rz)], ssum.at[pl.ds(srz, rz)])
        pltpu.sync_copy(zc_hbm.at[pl.ds(srz, rz)], scnt.at[pl.ds(srz, rz)])
        pltpu.sync_copy(on_hbm, ones_v)
        plsc.subcore_barrier()

        def body(i, carry):
            base = wid * np.int32(_PERW) + i * np.int32(_CH)
            pltpu.sync_copy(dst_hbm.at[pl.ds(base, _CH)], idx_v)
            pltpu.sync_copy(msgs_hbm.at[pl.ds(base, _CH)], rows_v)
            pltpu.sync_copy(rows_v, ssum.at[idx_v], add=True)
            pltpu.sync_copy(ones_v, scnt.at[idx_v], add=True)
            return carry

        lax.fori_loop(np.int32(0), np.int32(_NCHUNK), body, np.int32(0))
        plsc.subcore_barrier()
        obase = c * np.int32(_NP) + srz
        pltpu.sync_copy(ssum.at[pl.ds(srz, rz)], sums_out.at[pl.ds(obase, rz)])
        pltpu.sync_copy(scnt.at[pl.ds(srz, rz)], cnts_out.at[pl.ds(obase, rz)])

    return sk(msgs, dst_p, zsum, zcnt, ones)


def _combine_body(ps_a, ps_b, pc_a, pc_b, out):
    cnt = pc_a[:, 0:1] + pc_b[:, 0:1]
    out[...] = (ps_a[...] + ps_b[...]) / jnp.maximum(cnt, 1.0)


def _combine(psums, pcnts):
    nb = _NP // _TN
    return pl.pallas_call(
        _combine_body,
        grid=(_N // _TN,),
        in_specs=[
            pl.BlockSpec((_TN, 64), lambda i: (i, 0)),
            pl.BlockSpec((_TN, 64), lambda i: (i + nb, 0)),
            pl.BlockSpec((_TN, _CNTW), lambda i: (i, 0)),
            pl.BlockSpec((_TN, _CNTW), lambda i: (i + nb, 0)),
        ],
        out_specs=pl.BlockSpec((_TN, 64), lambda i: (i, 0)),
        out_shape=jax.ShapeDtypeStruct((_N, 64), jnp.float32),
        compiler_params=pltpu.CompilerParams(
            dimension_semantics=("parallel",)),
    )(psums, psums, pcnts, pcnts)


def kernel(node_attr, edge_index, edge_attr, edge_sh, W1, b1, W2, b2):
    out_dtype = jnp.result_type(node_attr.dtype, W1.dtype, W2.dtype,
                                b1.dtype, b2.dtype)
    with jax.enable_x64(False):
        pad = _EP - _E
        src_p = jnp.pad(edge_index[0].astype(jnp.int32), (0, pad))
        dst_p = jnp.pad(edge_index[1].astype(jnp.int32), (0, pad),
                        constant_values=_N)
        ea_p = jnp.pad(edge_attr.astype(jnp.float32), ((0, pad), (0, 0)))
        sh_p = jnp.pad(edge_sh.astype(jnp.float32), ((0, pad), (0, 0)))
        xs = _gather(node_attr.astype(jnp.float32), src_p)
        msgs = _dense(ea_p, xs, sh_p, W1.astype(jnp.float32),
                      b1.astype(jnp.float32).reshape(1, 16),
                      W2.astype(jnp.float32),
                      b2.astype(jnp.float32).reshape(1, 1024))
        zsum = jnp.zeros((_NP, 64), jnp.float32)
        zcnt = jnp.zeros((_NP, _CNTW), jnp.float32)
        ones = jnp.ones((_CH, _CNTW), jnp.float32)
        psums, pcnts = _scatter(msgs, dst_p, zsum, zcnt, ones)
        out = _combine(psums, pcnts)
    return out.astype(out_dtype)


# speedup vs baseline: 131.7993x; 1.0469x over previous
import functools

import numpy as np
import jax
import jax.numpy as jnp
from jax import lax
from jax.experimental import pallas as pl
from jax.experimental.pallas import tpu as pltpu
from jax.experimental.pallas import tpu_sc as plsc

_N = 10000
_E = 160000
_EP = 163840
_NP = 10400
_CH = 128
_NW = 32
_PERW = _EP // _NW
_NCHUNK = _PERW // _CH
_CNTW = 16
_TE = 256
_TN = 400
_UN = 4

_A0 = float(np.sqrt(1.0 / 32.0))
_A110 = float(np.sqrt(1.0 / 96.0))


def _build_consts():
    e1 = np.zeros((64, 48), np.float32)
    for u in range(16):
        for k in range(3):
            e1[16 + 3 * u + k, 16 * k + u] = 1.0
    r1 = np.zeros((16, 256), np.float32)
    for u in range(16):
        r1[u, u * 16:(u + 1) * 16] = 1.0
    s1 = np.zeros((256, 16), np.float32)
    for j in range(256):
        s1[j, j % 16] = 1.0
    g = np.zeros((48, 48), np.float32)
    for k in range(3):
        for w in range(16):
            g[16 * k + w, 3 * w + k] = 1.0
    return e1, r1, s1, g


_E1, _R1, _S1, _G = _build_consts()


def _dense_body(ea, xs, sh, w1, b1, w2, b2, e1, r1, s1, g, out):
    h = jnp.dot(ea[...], w1[...], preferred_element_type=jnp.float32) + b1[...]
    h = h * jax.nn.sigmoid(h)
    w = jnp.dot(h, w2[...], preferred_element_type=jnp.float32) + b2[...]
    x0 = xs[:, 0:16]
    x1c = jnp.dot(xs[...], e1[...], preferred_element_type=jnp.float32)
    sh0 = sh[:, 0:1]
    v000 = x0 * (sh0 * _A0)
    v110 = (x1c[:, 0:16] * sh[:, 1:2]
            + x1c[:, 16:32] * sh[:, 2:3]
            + x1c[:, 32:48] * sh[:, 3:4]) * _A110
    r = r1[...]
    s = s1[...]
    p0 = (jnp.dot(v000, r, preferred_element_type=jnp.float32) * w[:, 0:256]
          + jnp.dot(v110, r, preferred_element_type=jnp.float32) * w[:, 768:1024])
    out0 = jnp.dot(p0, s, preferred_element_type=jnp.float32)
    t011 = jnp.dot(jnp.dot(x0, r, preferred_element_type=jnp.float32) * w[:, 256:512],
                   s, preferred_element_type=jnp.float32)
    w101 = w[:, 512:768]
    o1 = []
    for k in range(3):
        vk = x1c[:, 16 * k:16 * (k + 1)] * (sh0 * _A0)
        o1k = (jnp.dot(jnp.dot(vk, r, preferred_element_type=jnp.float32) * w101,
                       s, preferred_element_type=jnp.float32)
               + t011 * (sh[:, 1 + k:2 + k] * _A0))
        o1.append(o1k)
    o1cat = jnp.dot(jnp.concatenate(o1, axis=1), g[...],
                    preferred_element_type=jnp.float32)
    out[...] = jnp.concatenate([out0, o1cat], axis=1)


def _dense(ea, xs, sh, w1, b1, w2, b2):
    grid = _EP // _TE
    return pl.pallas_call(
        _dense_body,
        grid=(grid,),
        in_specs=[
            pl.BlockSpec((_TE, 16), lambda i: (i, 0)),
            pl.BlockSpec((_TE, 64), lambda i: (i, 0)),
            pl.BlockSpec((_TE, 4), lambda i: (i, 0)),
            pl.BlockSpec((16, 16), lambda i: (0, 0)),
            pl.BlockSpec((1, 16), lambda i: (0, 0)),
            pl.BlockSpec((16, 1024), lambda i: (0, 0)),
            pl.BlockSpec((1, 1024), lambda i: (0, 0)),
            pl.BlockSpec((64, 48), lambda i: (0, 0)),
            pl.BlockSpec((16, 256), lambda i: (0, 0)),
            pl.BlockSpec((256, 16), lambda i: (0, 0)),
            pl.BlockSpec((48, 48), lambda i: (0, 0)),
        ],
        out_specs=pl.BlockSpec((_TE, 64), lambda i: (i, 0)),
        out_shape=jax.ShapeDtypeStruct((_EP, 64), jnp.float32),
        compiler_params=pltpu.CompilerParams(
            dimension_semantics=("parallel",)),
    )(ea, xs, sh, w1, b1, w2, b2,
      jnp.asarray(_E1), jnp.asarray(_R1), jnp.asarray(_S1), jnp.asarray(_G))


def _gather(nodes, src_p):
    mesh = plsc.VectorSubcoreMesh(core_axis_name="c", subcore_axis_name="s")

    @functools.partial(
        pl.kernel, mesh=mesh,
        out_type=jax.ShapeDtypeStruct((_EP, 64), jnp.float32),
        scratch_types=(
            [pltpu.VMEM((_CH,), jnp.int32) for _ in range(_UN)]
            + [pltpu.VMEM((_CH, 64), jnp.float32) for _ in range(_UN)]
            + [pltpu.SemaphoreType.DMA, pltpu.SemaphoreType.DMA]
        ),
        compiler_params=pltpu.CompilerParams(use_tc_tiling_on_sc=False),
    )
    def gk(nodes_hbm, src_hbm, out_hbm, *bufs):
        idxs = bufs[:_UN]
        rows = bufs[_UN:2 * _UN]
        sem, wsem = bufs[2 * _UN], bufs[2 * _UN + 1]
        wid = lax.axis_index("s") * 2 + lax.axis_index("c")

        def body(i, carry):
            base = wid * np.int32(_PERW) + i * np.int32(_UN * _CH)
            gh = []
            for b in range(_UN):
                off = base + np.int32(b * _CH)
                pltpu.sync_copy(src_hbm.at[pl.ds(off, _CH)], idxs[b])
                gh.append(pltpu.async_copy(nodes_hbm.at[idxs[b]], rows[b], sem))
            wh = []
            for b in range(_UN):
                off = base + np.int32(b * _CH)
                gh[b].wait()
                wh.append(pltpu.async_copy(rows[b], out_hbm.at[pl.ds(off, _CH)],
                                           wsem))
            for b in range(_UN):
                wh[b].wait()
            return carry

        lax.fori_loop(np.int32(0), np.int32(_NCHUNK // _UN), body, np.int32(0))

    return gk(nodes, src_p)


def _scatter(msgs, dst_p, zsum, zcnt, ones):
    mesh = plsc.VectorSubcoreMesh(core_axis_name="c", subcore_axis_name="s")

    @functools.partial(
        pl.kernel, mesh=mesh,
        out_type=(jax.ShapeDtypeStruct((2 * _NP, 64), jnp.float32),
                  jax.ShapeDtypeStruct((2 * _NP, _CNTW), jnp.float32)),
        scratch_types=(
            [pltpu.VMEM((_CH,), jnp.int32) for _ in range(_UN)]
            + [pltpu.VMEM((_CH, 64), jnp.float32) for _ in range(_UN)]
            + [
                pltpu.VMEM((_CH, _CNTW), jnp.float32),
                pltpu.VMEM_SHARED((_NP, 64), jnp.float32),
                pltpu.VMEM_SHARED((_NP, _CNTW), jnp.float32),
                pltpu.SemaphoreType.DMA,
            ]
        ),
        compiler_params=pltpu.CompilerParams(use_tc_tiling_on_sc=False),
    )
    def sk(msgs_hbm, dst_hbm, zs_hbm, zc_hbm, on_hbm,
           sums_out, cnts_out, *bufs):
        idxs = bufs[:_UN]
        rows = bufs[_UN:2 * _UN]
        ones_v, ssum, scnt, sem = bufs[2 * _UN:]
        c = lax.axis_index("c")
        s = lax.axis_index("s")
        wid = s * 2 + c
        rz = _NP // 16
        srz = s * np.int32(rz)
        pltpu.sync_copy(zs_hbm.at[pl.ds(srz, rz)], ssum.at[pl.ds(srz, rz)])
        pltpu.sync_copy(zc_hbm.at[pl.ds(srz, rz)], scnt.at[pl.ds(srz, rz)])
        pltpu.sync_copy(on_hbm, ones_v)
        plsc.subcore_barrier()

        def body(i, carry):
            base = wid * np.int32(_PERW) + i * np.int32(_UN * _CH)
            mh = []
            for b in range(_UN):
                off = base + np.int32(b * _CH)
                pltpu.sync_copy(dst_hbm.at[pl.ds(off, _CH)], idxs[b])
                mh.append(pltpu.async_copy(msgs_hbm.at[pl.ds(off, _CH)],
                                           rows[b], sem))
            for b in range(_UN):
                mh[b].wait()
                pltpu.sync_copy(rows[b], ssum.at[idxs[b]], add=True)
                pltpu.sync_copy(ones_v, scnt.at[idxs[b]], add=True)
            return carry

        lax.fori_loop(np.int32(0), np.int32(_NCHUNK // _UN), body, np.int32(0))
        plsc.subcore_barrier()
        obase = c * np.int32(_NP) + srz
        pltpu.sync_copy(ssum.at[pl.ds(srz, rz)], sums_out.at[pl.ds(obase, rz)])
        pltpu.sync_copy(scnt.at[pl.ds(srz, rz)], cnts_out.at[pl.ds(obase, rz)])

    return sk(msgs, dst_p, zsum, zcnt, ones)


def _combine_body(ps_a, ps_b, pc_a, pc_b, out):
    cnt = pc_a[:, 0:1] + pc_b[:, 0:1]
    out[...] = (ps_a[...] + ps_b[...]) / jnp.maximum(cnt, 1.0)


def _combine(psums, pcnts):
    nb = _NP // _TN
    return pl.pallas_call(
        _combine_body,
        grid=(_N // _TN,),
        in_specs=[
            pl.BlockSpec((_TN, 64), lambda i: (i, 0)),
            pl.BlockSpec((_TN, 64), lambda i: (i + nb, 0)),
            pl.BlockSpec((_TN, _CNTW), lambda i: (i, 0)),
            pl.BlockSpec((_TN, _CNTW), lambda i: (i + nb, 0)),
        ],
        out_specs=pl.BlockSpec((_TN, 64), lambda i: (i, 0)),
        out_shape=jax.ShapeDtypeStruct((_N, 64), jnp.float32),
        compiler_params=pltpu.CompilerParams(
            dimension_semantics=("parallel",)),
    )(psums, psums, pcnts, pcnts)


def kernel(node_attr, edge_index, edge_attr, edge_sh, W1, b1, W2, b2):
    out_dtype = jnp.result_type(node_attr.dtype, W1.dtype, W2.dtype,
                                b1.dtype, b2.dtype)
    with jax.enable_x64(False):
        pad = _EP - _E
        src_p = jnp.pad(edge_index[0].astype(jnp.int32), (0, pad))
        dst_p = jnp.pad(edge_index[1].astype(jnp.int32), (0, pad),
                        constant_values=_N)
        ea_p = jnp.pad(edge_attr.astype(jnp.float32), ((0, pad), (0, 0)))
        sh_p = jnp.pad(edge_sh.astype(jnp.float32), ((0, pad), (0, 0)))
        xs = _gather(node_attr.astype(jnp.float32), src_p)
        msgs = _dense(ea_p, xs, sh_p, W1.astype(jnp.float32),
                      b1.astype(jnp.float32).reshape(1, 16),
                      W2.astype(jnp.float32),
                      b2.astype(jnp.float32).reshape(1, 1024))
        zsum = jnp.zeros((_NP, 64), jnp.float32)
        zcnt = jnp.zeros((_NP, _CNTW), jnp.float32)
        ones = jnp.ones((_CH, _CNTW), jnp.float32)
        psums, pcnts = _scatter(msgs, dst_p, zsum, zcnt, ones)
        out = _combine(psums, pcnts)
    return out.astype(out_dtype)


# speedup vs baseline: 163.3391x; 1.2393x over previous
import functools

import numpy as np
import jax
import jax.numpy as jnp
from jax import lax
from jax.experimental import pallas as pl
from jax.experimental.pallas import tpu as pltpu
from jax.experimental.pallas import tpu_sc as plsc

_N = 10000
_E = 160000
_EP = 163840
_NP = 10400
_CH = 128
_NW = 32
_PERW = _EP // _NW
_NCHUNK = _PERW // _CH
_CNTW = 16
_TE = 512
_TN = 400
_UN = 4

_A0 = float(np.sqrt(1.0 / 32.0))
_A110 = float(np.sqrt(1.0 / 96.0))


def _build_consts():
    e1 = np.zeros((64, 48), np.float32)
    for u in range(16):
        for k in range(3):
            e1[16 + 3 * u + k, 16 * k + u] = 1.0
    r1 = np.zeros((16, 256), np.float32)
    for u in range(16):
        r1[u, u * 16:(u + 1) * 16] = 1.0
    s1 = np.zeros((256, 16), np.float32)
    for j in range(256):
        s1[j, j % 16] = 1.0
    g = np.zeros((48, 48), np.float32)
    for k in range(3):
        for w in range(16):
            g[16 * k + w, 3 * w + k] = 1.0
    return e1, r1, s1, g


_E1, _R1, _S1, _G = _build_consts()


def _dense_body(ea, xs, sh, w1, b1, w2, b2, e1, r1, s1, g, out):
    h = jnp.dot(ea[...], w1[...], preferred_element_type=jnp.float32) + b1[...]
    h = h * jax.nn.sigmoid(h)
    w = jnp.dot(h, w2[...], preferred_element_type=jnp.float32) + b2[...]
    x0 = xs[:, 0:16]
    x1c = jnp.dot(xs[...], e1[...], preferred_element_type=jnp.float32)
    sh0 = sh[:, 0:1]
    v000 = x0 * (sh0 * _A0)
    v110 = (x1c[:, 0:16] * sh[:, 1:2]
            + x1c[:, 16:32] * sh[:, 2:3]
            + x1c[:, 32:48] * sh[:, 3:4]) * _A110
    r = r1[...]
    s = s1[...]
    p0 = (jnp.dot(v000, r, preferred_element_type=jnp.float32) * w[:, 0:256]
          + jnp.dot(v110, r, preferred_element_type=jnp.float32) * w[:, 768:1024])
    out0 = jnp.dot(p0, s, preferred_element_type=jnp.float32)
    t011 = jnp.dot(jnp.dot(x0, r, preferred_element_type=jnp.float32) * w[:, 256:512],
                   s, preferred_element_type=jnp.float32)
    w101 = w[:, 512:768]
    o1 = []
    for k in range(3):
        vk = x1c[:, 16 * k:16 * (k + 1)] * (sh0 * _A0)
        o1k = (jnp.dot(jnp.dot(vk, r, preferred_element_type=jnp.float32) * w101,
                       s, preferred_element_type=jnp.float32)
               + t011 * (sh[:, 1 + k:2 + k] * _A0))
        o1.append(o1k)
    o1cat = jnp.dot(jnp.concatenate(o1, axis=1), g[...],
                    preferred_element_type=jnp.float32)
    out[...] = jnp.concatenate([out0, o1cat], axis=1)


def _dense(ea, xs, sh, w1, b1, w2, b2):
    grid = _EP // _TE
    return pl.pallas_call(
        _dense_body,
        grid=(grid,),
        in_specs=[
            pl.BlockSpec((_TE, 16), lambda i: (i, 0)),
            pl.BlockSpec((_TE, 64), lambda i: (i, 0)),
            pl.BlockSpec((_TE, 4), lambda i: (i, 0)),
            pl.BlockSpec((16, 16), lambda i: (0, 0)),
            pl.BlockSpec((1, 16), lambda i: (0, 0)),
            pl.BlockSpec((16, 1024), lambda i: (0, 0)),
            pl.BlockSpec((1, 1024), lambda i: (0, 0)),
            pl.BlockSpec((64, 48), lambda i: (0, 0)),
            pl.BlockSpec((16, 256), lambda i: (0, 0)),
            pl.BlockSpec((256, 16), lambda i: (0, 0)),
            pl.BlockSpec((48, 48), lambda i: (0, 0)),
        ],
        out_specs=pl.BlockSpec((_TE, 64), lambda i: (i, 0)),
        out_shape=jax.ShapeDtypeStruct((_EP, 64), jnp.float32),
        compiler_params=pltpu.CompilerParams(
            dimension_semantics=("parallel",)),
    )(ea, xs, sh, w1, b1, w2, b2,
      jnp.asarray(_E1), jnp.asarray(_R1), jnp.asarray(_S1), jnp.asarray(_G))


def _gather(nodes, src_p):
    mesh = plsc.VectorSubcoreMesh(core_axis_name="c", subcore_axis_name="s")

    @functools.partial(
        pl.kernel, mesh=mesh,
        out_type=jax.ShapeDtypeStruct((_EP, 64), jnp.float32),
        scratch_types=(
            [pltpu.VMEM((_CH,), jnp.int32) for _ in range(_UN)]
            + [pltpu.VMEM((_CH, 64), jnp.float32) for _ in range(_UN)]
            + [pltpu.SemaphoreType.DMA, pltpu.SemaphoreType.DMA]
        ),
        compiler_params=pltpu.CompilerParams(use_tc_tiling_on_sc=False),
    )
    def gk(nodes_hbm, src_hbm, out_hbm, *bufs):
        idxs = bufs[:_UN]
        rows = bufs[_UN:2 * _UN]
        sem, wsem = bufs[2 * _UN], bufs[2 * _UN + 1]
        wid = lax.axis_index("s") * 2 + lax.axis_index("c")

        def body(i, carry):
            base = wid * np.int32(_PERW) + i * np.int32(_UN * _CH)
            gh = []
            for b in range(_UN):
                off = base + np.int32(b * _CH)
                pltpu.sync_copy(src_hbm.at[pl.ds(off, _CH)], idxs[b])
                gh.append(pltpu.async_copy(nodes_hbm.at[idxs[b]], rows[b], sem))
            wh = []
            for b in range(_UN):
                off = base + np.int32(b * _CH)
                gh[b].wait()
                wh.append(pltpu.async_copy(rows[b], out_hbm.at[pl.ds(off, _CH)],
                                           wsem))
            for b in range(_UN):
                wh[b].wait()
            return carry

        lax.fori_loop(np.int32(0), np.int32(_NCHUNK // _UN), body, np.int32(0))

    return gk(nodes, src_p)


def _scatter(msgs, dst_p, zsum, zcnt, ones):
    mesh = plsc.VectorSubcoreMesh(core_axis_name="c", subcore_axis_name="s")

    @functools.partial(
        pl.kernel, mesh=mesh,
        out_type=(jax.ShapeDtypeStruct((2 * _NP, 64), jnp.float32),
                  jax.ShapeDtypeStruct((2 * _NP, _CNTW), jnp.float32)),
        scratch_types=(
            [pltpu.VMEM((_CH,), jnp.int32) for _ in range(_UN)]
            + [pltpu.VMEM((_CH, 64), jnp.float32) for _ in range(_UN)]
            + [
                pltpu.VMEM((_CH, _CNTW), jnp.float32),
                pltpu.VMEM_SHARED((_NP, 64), jnp.float32),
                pltpu.VMEM_SHARED((_NP, _CNTW), jnp.float32),
                pltpu.SemaphoreType.DMA,
            ]
        ),
        compiler_params=pltpu.CompilerParams(use_tc_tiling_on_sc=False),
    )
    def sk(msgs_hbm, dst_hbm, zs_hbm, zc_hbm, on_hbm,
           sums_out, cnts_out, *bufs):
        idxs = bufs[:_UN]
        rows = bufs[_UN:2 * _UN]
        ones_v, ssum, scnt, sem = bufs[2 * _UN:]
        c = lax.axis_index("c")
        s = lax.axis_index("s")
        wid = s * 2 + c
        rz = _NP // 16
        srz = s * np.int32(rz)
        pltpu.sync_copy(zs_hbm.at[pl.ds(srz, rz)], ssum.at[pl.ds(srz, rz)])
        pltpu.sync_copy(zc_hbm.at[pl.ds(srz, rz)], scnt.at[pl.ds(srz, rz)])
        pltpu.sync_copy(on_hbm, ones_v)
        plsc.subcore_barrier()

        def body(i, carry):
            base = wid * np.int32(_PERW) + i * np.int32(_UN * _CH)
            mh = []
            for b in range(_UN):
                off = base + np.int32(b * _CH)
                pltpu.sync_copy(dst_hbm.at[pl.ds(off, _CH)], idxs[b])
                mh.append(pltpu.async_copy(msgs_hbm.at[pl.ds(off, _CH)],
                                           rows[b], sem))
            for b in range(_UN):
                mh[b].wait()
                pltpu.sync_copy(rows[b], ssum.at[idxs[b]], add=True)
                pltpu.sync_copy(ones_v, scnt.at[idxs[b]], add=True)
            return carry

        lax.fori_loop(np.int32(0), np.int32(_NCHUNK // _UN), body, np.int32(0))
        plsc.subcore_barrier()
        obase = c * np.int32(_NP) + srz
        pltpu.sync_copy(ssum.at[pl.ds(srz, rz)], sums_out.at[pl.ds(obase, rz)])
        pltpu.sync_copy(scnt.at[pl.ds(srz, rz)], cnts_out.at[pl.ds(obase, rz)])

    return sk(msgs, dst_p, zsum, zcnt, ones)


def _combine_body(ps_a, ps_b, pc_a, pc_b, out):
    cnt = pc_a[:, 0:1] + pc_b[:, 0:1]
    out[...] = (ps_a[...] + ps_b[...]) / jnp.maximum(cnt, 1.0)


def _combine(psums, pcnts):
    nb = _NP // _TN
    return pl.pallas_call(
        _combine_body,
        grid=(_N // _TN,),
        in_specs=[
            pl.BlockSpec((_TN, 64), lambda i: (i, 0)),
            pl.BlockSpec((_TN, 64), lambda i: (i + nb, 0)),
            pl.BlockSpec((_TN, _CNTW), lambda i: (i, 0)),
            pl.BlockSpec((_TN, _CNTW), lambda i: (i + nb, 0)),
        ],
        out_specs=pl.BlockSpec((_TN, 64), lambda i: (i, 0)),
        out_shape=jax.ShapeDtypeStruct((_N, 64), jnp.float32),
        compiler_params=pltpu.CompilerParams(
            dimension_semantics=("parallel",)),
    )(psums, psums, pcnts, pcnts)


def kernel(node_attr, edge_index, edge_attr, edge_sh, W1, b1, W2, b2):
    out_dtype = jnp.result_type(node_attr.dtype, W1.dtype, W2.dtype,
                                b1.dtype, b2.dtype)
    with jax.enable_x64(False):
        pad = _EP - _E
        src_p = jnp.pad(edge_index[0].astype(jnp.int32), (0, pad))
        dst_p = jnp.pad(edge_index[1].astype(jnp.int32), (0, pad),
                        constant_values=_N)
        ea_p = jnp.pad(edge_attr.astype(jnp.float32), ((0, pad), (0, 0)))
        sh_p = jnp.pad(edge_sh.astype(jnp.float32), ((0, pad), (0, 0)))
        xs = _gather(node_attr.astype(jnp.float32), src_p)
        msgs = _dense(ea_p, xs, sh_p, W1.astype(jnp.float32),
                      b1.astype(jnp.float32).reshape(1, 16),
                      W2.astype(jnp.float32),
                      b2.astype(jnp.float32).reshape(1, 1024))
        zsum = jnp.zeros((_NP, 64), jnp.float32)
        zcnt = jnp.zeros((_NP, _CNTW), jnp.float32)
        ones = jnp.ones((_CH, _CNTW), jnp.float32)
        psums, pcnts = _scatter(msgs, dst_p, zsum, zcnt, ones)
        out = _combine(psums, pcnts)
    return out.astype(out_dtype)


# speedup vs baseline: 189.2458x; 1.1586x over previous
import functools

import numpy as np
import jax
import jax.numpy as jnp
from jax import lax
from jax.experimental import pallas as pl
from jax.experimental.pallas import tpu as pltpu
from jax.experimental.pallas import tpu_sc as plsc

_N = 10000
_E = 160000
_EP = 163840
_NP = 10400
_CH = 128
_NW = 32
_PERW = _EP // _NW
_NCHUNK = _PERW // _CH
_CNTW = 16
_TE = 1024
_TN = 400
_UN = 4

_A0 = float(np.sqrt(1.0 / 32.0))
_A110 = float(np.sqrt(1.0 / 96.0))


def _build_consts():
    e1 = np.zeros((64, 48), np.float32)
    for u in range(16):
        for k in range(3):
            e1[16 + 3 * u + k, 16 * k + u] = 1.0
    r1 = np.zeros((16, 256), np.float32)
    for u in range(16):
        r1[u, u * 16:(u + 1) * 16] = 1.0
    s1 = np.zeros((256, 16), np.float32)
    for j in range(256):
        s1[j, j % 16] = 1.0
    g = np.zeros((48, 48), np.float32)
    for k in range(3):
        for w in range(16):
            g[16 * k + w, 3 * w + k] = 1.0
    return e1, r1, s1, g


_E1, _R1, _S1, _G = _build_consts()


def _dense_body(ea, xs, sh, w1, b1, w2, b2, e1, r1, s1, g, out):
    h = jnp.dot(ea[...], w1[...], preferred_element_type=jnp.float32) + b1[...]
    h = h * jax.nn.sigmoid(h)
    w = jnp.dot(h, w2[...], preferred_element_type=jnp.float32) + b2[...]
    x0 = xs[:, 0:16]
    x1c = jnp.dot(xs[...], e1[...], preferred_element_type=jnp.float32)
    sh0 = sh[:, 0:1]
    v000 = x0 * (sh0 * _A0)
    v110 = (x1c[:, 0:16] * sh[:, 1:2]
            + x1c[:, 16:32] * sh[:, 2:3]
            + x1c[:, 32:48] * sh[:, 3:4]) * _A110
    r = r1[...]
    s = s1[...]
    p0 = (jnp.dot(v000, r, preferred_element_type=jnp.float32) * w[:, 0:256]
          + jnp.dot(v110, r, preferred_element_type=jnp.float32) * w[:, 768:1024])
    out0 = jnp.dot(p0, s, preferred_element_type=jnp.float32)
    t011 = jnp.dot(jnp.dot(x0, r, preferred_element_type=jnp.float32) * w[:, 256:512],
                   s, preferred_element_type=jnp.float32)
    w101 = w[:, 512:768]
    o1 = []
    for k in range(3):
        vk = x1c[:, 16 * k:16 * (k + 1)] * (sh0 * _A0)
        o1k = (jnp.dot(jnp.dot(vk, r, preferred_element_type=jnp.float32) * w101,
                       s, preferred_element_type=jnp.float32)
               + t011 * (sh[:, 1 + k:2 + k] * _A0))
        o1.append(o1k)
    o1cat = jnp.dot(jnp.concatenate(o1, axis=1), g[...],
                    preferred_element_type=jnp.float32)
    out[...] = jnp.concatenate([out0, o1cat], axis=1)


def _dense(ea, xs, sh, w1, b1, w2, b2):
    grid = _EP // _TE
    return pl.pallas_call(
        _dense_body,
        grid=(grid,),
        in_specs=[
            pl.BlockSpec((_TE, 16), lambda i: (i, 0)),
            pl.BlockSpec((_TE, 64), lambda i: (i, 0)),
            pl.BlockSpec((_TE, 4), lambda i: (i, 0)),
            pl.BlockSpec((16, 16), lambda i: (0, 0)),
            pl.BlockSpec((1, 16), lambda i: (0, 0)),
            pl.BlockSpec((16, 1024), lambda i: (0, 0)),
            pl.BlockSpec((1, 1024), lambda i: (0, 0)),
            pl.BlockSpec((64, 48), lambda i: (0, 0)),
            pl.BlockSpec((16, 256), lambda i: (0, 0)),
            pl.BlockSpec((256, 16), lambda i: (0, 0)),
            pl.BlockSpec((48, 48), lambda i: (0, 0)),
        ],
        out_specs=pl.BlockSpec((_TE, 64), lambda i: (i, 0)),
        out_shape=jax.ShapeDtypeStruct((_EP, 64), jnp.float32),
        compiler_params=pltpu.CompilerParams(
            dimension_semantics=("parallel",)),
    )(ea, xs, sh, w1, b1, w2, b2,
      jnp.asarray(_E1), jnp.asarray(_R1), jnp.asarray(_S1), jnp.asarray(_G))


def _gather(nodes, src_p):
    mesh = plsc.VectorSubcoreMesh(core_axis_name="c", subcore_axis_name="s")

    @functools.partial(
        pl.kernel, mesh=mesh,
        out_type=jax.ShapeDtypeStruct((_EP, 64), jnp.float32),
        scratch_types=(
            [pltpu.VMEM((_CH,), jnp.int32) for _ in range(_UN)]
            + [pltpu.VMEM((_CH, 64), jnp.float32) for _ in range(_UN)]
            + [pltpu.SemaphoreType.DMA, pltpu.SemaphoreType.DMA]
        ),
        compiler_params=pltpu.CompilerParams(use_tc_tiling_on_sc=False),
    )
    def gk(nodes_hbm, src_hbm, out_hbm, *bufs):
        idxs = bufs[:_UN]
        rows = bufs[_UN:2 * _UN]
        sem, wsem = bufs[2 * _UN], bufs[2 * _UN + 1]
        wid = lax.axis_index("s") * 2 + lax.axis_index("c")

        def body(i, carry):
            base = wid * np.int32(_PERW) + i * np.int32(_UN * _CH)
            gh = []
            for b in range(_UN):
                off = base + np.int32(b * _CH)
                pltpu.sync_copy(src_hbm.at[pl.ds(off, _CH)], idxs[b])
                gh.append(pltpu.async_copy(nodes_hbm.at[idxs[b]], rows[b], sem))
            wh = []
            for b in range(_UN):
                off = base + np.int32(b * _CH)
                gh[b].wait()
                wh.append(pltpu.async_copy(rows[b], out_hbm.at[pl.ds(off, _CH)],
                                           wsem))
            for b in range(_UN):
                wh[b].wait()
            return carry

        lax.fori_loop(np.int32(0), np.int32(_NCHUNK // _UN), body, np.int32(0))

    return gk(nodes, src_p)


def _scatter(msgs, dst_p, zsum, zcnt, ones):
    mesh = plsc.VectorSubcoreMesh(core_axis_name="c", subcore_axis_name="s")

    @functools.partial(
        pl.kernel, mesh=mesh,
        out_type=(jax.ShapeDtypeStruct((2 * _NP, 64), jnp.float32),
                  jax.ShapeDtypeStruct((2 * _NP, _CNTW), jnp.float32)),
        scratch_types=(
            [pltpu.VMEM((_CH,), jnp.int32) for _ in range(_UN)]
            + [pltpu.VMEM((_CH, 64), jnp.float32) for _ in range(_UN)]
            + [
                pltpu.VMEM((_CH, _CNTW), jnp.float32),
                pltpu.VMEM_SHARED((_NP, 64), jnp.float32),
                pltpu.VMEM_SHARED((_NP, _CNTW), jnp.float32),
                pltpu.SemaphoreType.DMA,
            ]
        ),
        compiler_params=pltpu.CompilerParams(use_tc_tiling_on_sc=False),
    )
    def sk(msgs_hbm, dst_hbm, zs_hbm, zc_hbm, on_hbm,
           sums_out, cnts_out, *bufs):
        idxs = bufs[:_UN]
        rows = bufs[_UN:2 * _UN]
        ones_v, ssum, scnt, sem = bufs[2 * _UN:]
        c = lax.axis_index("c")
        s = lax.axis_index("s")
        wid = s * 2 + c
        rz = _NP // 16
        srz = s * np.int32(rz)
        pltpu.sync_copy(zs_hbm.at[pl.ds(srz, rz)], ssum.at[pl.ds(srz, rz)])
        pltpu.sync_copy(zc_hbm.at[pl.ds(srz, rz)], scnt.at[pl.ds(srz, rz)])
        pltpu.sync_copy(on_hbm, ones_v)
        plsc.subcore_barrier()

        def body(i, carry):
            base = wid * np.int32(_PERW) + i * np.int32(_UN * _CH)
            mh = []
            for b in range(_UN):
                off = base + np.int32(b * _CH)
                pltpu.sync_copy(dst_hbm.at[pl.ds(off, _CH)], idxs[b])
                mh.append(pltpu.async_copy(msgs_hbm.at[pl.ds(off, _CH)],
                                           rows[b], sem))
            for b in range(_UN):
                mh[b].wait()
                pltpu.sync_copy(rows[b], ssum.at[idxs[b]], add=True)
                pltpu.sync_copy(ones_v, scnt.at[idxs[b]], add=True)
            return carry

        lax.fori_loop(np.int32(0), np.int32(_NCHUNK // _UN), body, np.int32(0))
        plsc.subcore_barrier()
        obase = c * np.int32(_NP) + srz
        pltpu.sync_copy(ssum.at[pl.ds(srz, rz)], sums_out.at[pl.ds(obase, rz)])
        pltpu.sync_copy(scnt.at[pl.ds(srz, rz)], cnts_out.at[pl.ds(obase, rz)])

    return sk(msgs, dst_p, zsum, zcnt, ones)


def _combine_body(ps_a, ps_b, pc_a, pc_b, out):
    cnt = pc_a[:, 0:1] + pc_b[:, 0:1]
    out[...] = (ps_a[...] + ps_b[...]) / jnp.maximum(cnt, 1.0)


def _combine(psums, pcnts):
    nb = _NP // _TN
    return pl.pallas_call(
        _combine_body,
        grid=(_N // _TN,),
        in_specs=[
            pl.BlockSpec((_TN, 64), lambda i: (i, 0)),
            pl.BlockSpec((_TN, 64), lambda i: (i + nb, 0)),
            pl.BlockSpec((_TN, _CNTW), lambda i: (i, 0)),
            pl.BlockSpec((_TN, _CNTW), lambda i: (i + nb, 0)),
        ],
        out_specs=pl.BlockSpec((_TN, 64), lambda i: (i, 0)),
        out_shape=jax.ShapeDtypeStruct((_N, 64), jnp.float32),
        compiler_params=pltpu.CompilerParams(
            dimension_semantics=("parallel",)),
    )(psums, psums, pcnts, pcnts)


def kernel(node_attr, edge_index, edge_attr, edge_sh, W1, b1, W2, b2):
    out_dtype = jnp.result_type(node_attr.dtype, W1.dtype, W2.dtype,
                                b1.dtype, b2.dtype)
    with jax.enable_x64(False):
        pad = _EP - _E
        src_p = jnp.pad(edge_index[0].astype(jnp.int32), (0, pad))
        dst_p = jnp.pad(edge_index[1].astype(jnp.int32), (0, pad),
                        constant_values=_N)
        ea_p = jnp.pad(edge_attr.astype(jnp.float32), ((0, pad), (0, 0)))
        sh_p = jnp.pad(edge_sh.astype(jnp.float32), ((0, pad), (0, 0)))
        xs = _gather(node_attr.astype(jnp.float32), src_p)
        msgs = _dense(ea_p, xs, sh_p, W1.astype(jnp.float32),
                      b1.astype(jnp.float32).reshape(1, 16),
                      W2.astype(jnp.float32),
                      b2.astype(jnp.float32).reshape(1, 1024))
        zsum = jnp.zeros((_NP, 64), jnp.float32)
        zcnt = jnp.zeros((_NP, _CNTW), jnp.float32)
        ones = jnp.ones((_CH, _CNTW), jnp.float32)
        psums, pcnts = _scatter(msgs, dst_p, zsum, zcnt, ones)
        out = _combine(psums, pcnts)
    return out.astype(out_dtype)


# speedup vs baseline: 197.3419x; 1.0428x over previous
import functools

import numpy as np
import jax
import jax.numpy as jnp
from jax import lax
from jax.experimental import pallas as pl
from jax.experimental.pallas import tpu as pltpu
from jax.experimental.pallas import tpu_sc as plsc

_N = 10000
_E = 160000
_EP = 163840
_NP = 10400
_CH = 128
_NW = 32
_PERW = _EP // _NW
_NCHUNK = _PERW // _CH
_CNTW = 16
_TE = 2048
_TN = 400
_UN = 4

_A0 = float(np.sqrt(1.0 / 32.0))
_A110 = float(np.sqrt(1.0 / 96.0))


def _build_consts():
    e1 = np.zeros((64, 48), np.float32)
    for u in range(16):
        for k in range(3):
            e1[16 + 3 * u + k, 16 * k + u] = 1.0
    r1 = np.zeros((16, 256), np.float32)
    for u in range(16):
        r1[u, u * 16:(u + 1) * 16] = 1.0
    s1 = np.zeros((256, 16), np.float32)
    for j in range(256):
        s1[j, j % 16] = 1.0
    g = np.zeros((48, 48), np.float32)
    for k in range(3):
        for w in range(16):
            g[16 * k + w, 3 * w + k] = 1.0
    return e1, r1, s1, g


_E1, _R1, _S1, _G = _build_consts()


def _dense_body(ea, xs, sh, w1, b1, w2, b2, e1, r1, s1, g, out):
    h = jnp.dot(ea[...], w1[...], preferred_element_type=jnp.float32) + b1[...]
    h = h * jax.nn.sigmoid(h)
    w = jnp.dot(h, w2[...], preferred_element_type=jnp.float32) + b2[...]
    x0 = xs[:, 0:16]
    x1c = jnp.dot(xs[...], e1[...], preferred_element_type=jnp.float32)
    sh0 = sh[:, 0:1]
    v000 = x0 * (sh0 * _A0)
    v110 = (x1c[:, 0:16] * sh[:, 1:2]
            + x1c[:, 16:32] * sh[:, 2:3]
            + x1c[:, 32:48] * sh[:, 3:4]) * _A110
    r = r1[...]
    s = s1[...]
    p0 = (jnp.dot(v000, r, preferred_element_type=jnp.float32) * w[:, 0:256]
          + jnp.dot(v110, r, preferred_element_type=jnp.float32) * w[:, 768:1024])
    out0 = jnp.dot(p0, s, preferred_element_type=jnp.float32)
    t011 = jnp.dot(jnp.dot(x0, r, preferred_element_type=jnp.float32) * w[:, 256:512],
                   s, preferred_element_type=jnp.float32)
    w101 = w[:, 512:768]
    o1 = []
    for k in range(3):
        vk = x1c[:, 16 * k:16 * (k + 1)] * (sh0 * _A0)
        o1k = (jnp.dot(jnp.dot(vk, r, preferred_element_type=jnp.float32) * w101,
                       s, preferred_element_type=jnp.float32)
               + t011 * (sh[:, 1 + k:2 + k] * _A0))
        o1.append(o1k)
    o1cat = jnp.dot(jnp.concatenate(o1, axis=1), g[...],
                    preferred_element_type=jnp.float32)
    out[...] = jnp.concatenate([out0, o1cat], axis=1)


def _dense(ea, xs, sh, w1, b1, w2, b2):
    grid = _EP // _TE
    return pl.pallas_call(
        _dense_body,
        grid=(grid,),
        in_specs=[
            pl.BlockSpec((_TE, 16), lambda i: (i, 0)),
            pl.BlockSpec((_TE, 64), lambda i: (i, 0)),
            pl.BlockSpec((_TE, 4), lambda i: (i, 0)),
            pl.BlockSpec((16, 16), lambda i: (0, 0)),
            pl.BlockSpec((1, 16), lambda i: (0, 0)),
            pl.BlockSpec((16, 1024), lambda i: (0, 0)),
            pl.BlockSpec((1, 1024), lambda i: (0, 0)),
            pl.BlockSpec((64, 48), lambda i: (0, 0)),
            pl.BlockSpec((16, 256), lambda i: (0, 0)),
            pl.BlockSpec((256, 16), lambda i: (0, 0)),
            pl.BlockSpec((48, 48), lambda i: (0, 0)),
        ],
        out_specs=pl.BlockSpec((_TE, 64), lambda i: (i, 0)),
        out_shape=jax.ShapeDtypeStruct((_EP, 64), jnp.float32),
        compiler_params=pltpu.CompilerParams(
            dimension_semantics=("parallel",)),
    )(ea, xs, sh, w1, b1, w2, b2,
      jnp.asarray(_E1), jnp.asarray(_R1), jnp.asarray(_S1), jnp.asarray(_G))


def _gather(nodes, src_p):
    mesh = plsc.VectorSubcoreMesh(core_axis_name="c", subcore_axis_name="s")

    @functools.partial(
        pl.kernel, mesh=mesh,
        out_type=jax.ShapeDtypeStruct((_EP, 64), jnp.float32),
        scratch_types=(
            [pltpu.VMEM((_CH,), jnp.int32) for _ in range(_UN)]
            + [pltpu.VMEM((_CH, 64), jnp.float32) for _ in range(_UN)]
            + [pltpu.SemaphoreType.DMA, pltpu.SemaphoreType.DMA]
        ),
        compiler_params=pltpu.CompilerParams(use_tc_tiling_on_sc=False),
    )
    def gk(nodes_hbm, src_hbm, out_hbm, *bufs):
        idxs = bufs[:_UN]
        rows = bufs[_UN:2 * _UN]
        sem, wsem = bufs[2 * _UN], bufs[2 * _UN + 1]
        wid = lax.axis_index("s") * 2 + lax.axis_index("c")

        def body(i, carry):
            base = wid * np.int32(_PERW) + i * np.int32(_UN * _CH)
            gh = []
            for b in range(_UN):
                off = base + np.int32(b * _CH)
                pltpu.sync_copy(src_hbm.at[pl.ds(off, _CH)], idxs[b])
                gh.append(pltpu.async_copy(nodes_hbm.at[idxs[b]], rows[b], sem))
            wh = []
            for b in range(_UN):
                off = base + np.int32(b * _CH)
                gh[b].wait()
                wh.append(pltpu.async_copy(rows[b], out_hbm.at[pl.ds(off, _CH)],
                                           wsem))
            for b in range(_UN):
                wh[b].wait()
            return carry

        lax.fori_loop(np.int32(0), np.int32(_NCHUNK // _UN), body, np.int32(0))

    return gk(nodes, src_p)


def _scatter(msgs, dst_p, zsum, zcnt, ones):
    mesh = plsc.VectorSubcoreMesh(core_axis_name="c", subcore_axis_name="s")

    @functools.partial(
        pl.kernel, mesh=mesh,
        out_type=(jax.ShapeDtypeStruct((2 * _NP, 64), jnp.float32),
                  jax.ShapeDtypeStruct((2 * _NP, _CNTW), jnp.float32)),
        scratch_types=(
            [pltpu.VMEM((_CH,), jnp.int32) for _ in range(_UN)]
            + [pltpu.VMEM((_CH, 64), jnp.float32) for _ in range(_UN)]
            + [
                pltpu.VMEM((_CH, _CNTW), jnp.float32),
                pltpu.VMEM_SHARED((_NP, 64), jnp.float32),
                pltpu.VMEM_SHARED((_NP, _CNTW), jnp.float32),
                pltpu.SemaphoreType.DMA,
            ]
        ),
        compiler_params=pltpu.CompilerParams(use_tc_tiling_on_sc=False),
    )
    def sk(msgs_hbm, dst_hbm, zs_hbm, zc_hbm, on_hbm,
           sums_out, cnts_out, *bufs):
        idxs = bufs[:_UN]
        rows = bufs[_UN:2 * _UN]
        ones_v, ssum, scnt, sem = bufs[2 * _UN:]
        c = lax.axis_index("c")
        s = lax.axis_index("s")
        wid = s * 2 + c
        rz = _NP // 16
        srz = s * np.int32(rz)
        pltpu.sync_copy(zs_hbm.at[pl.ds(srz, rz)], ssum.at[pl.ds(srz, rz)])
        pltpu.sync_copy(zc_hbm.at[pl.ds(srz, rz)], scnt.at[pl.ds(srz, rz)])
        pltpu.sync_copy(on_hbm, ones_v)
        plsc.subcore_barrier()

        def body(i, carry):
            base = wid * np.int32(_PERW) + i * np.int32(_UN * _CH)
            mh = []
            for b in range(_UN):
                off = base + np.int32(b * _CH)
                pltpu.sync_copy(dst_hbm.at[pl.ds(off, _CH)], idxs[b])
                mh.append(pltpu.async_copy(msgs_hbm.at[pl.ds(off, _CH)],
                                           rows[b], sem))
            for b in range(_UN):
                mh[b].wait()
                pltpu.sync_copy(rows[b], ssum.at[idxs[b]], add=True)
                pltpu.sync_copy(ones_v, scnt.at[idxs[b]], add=True)
            return carry

        lax.fori_loop(np.int32(0), np.int32(_NCHUNK // _UN), body, np.int32(0))
        plsc.subcore_barrier()
        obase = c * np.int32(_NP) + srz
        pltpu.sync_copy(ssum.at[pl.ds(srz, rz)], sums_out.at[pl.ds(obase, rz)])
        pltpu.sync_copy(scnt.at[pl.ds(srz, rz)], cnts_out.at[pl.ds(obase, rz)])

    return sk(msgs, dst_p, zsum, zcnt, ones)


def _combine_body(ps_a, ps_b, pc_a, pc_b, out):
    cnt = pc_a[:, 0:1] + pc_b[:, 0:1]
    out[...] = (ps_a[...] + ps_b[...]) / jnp.maximum(cnt, 1.0)


def _combine(psums, pcnts):
    nb = _NP // _TN
    return pl.pallas_call(
        _combine_body,
        grid=(_N // _TN,),
        in_specs=[
            pl.BlockSpec((_TN, 64), lambda i: (i, 0)),
            pl.BlockSpec((_TN, 64), lambda i: (i + nb, 0)),
            pl.BlockSpec((_TN, _CNTW), lambda i: (i, 0)),
            pl.BlockSpec((_TN, _CNTW), lambda i: (i + nb, 0)),
        ],
        out_specs=pl.BlockSpec((_TN, 64), lambda i: (i, 0)),
        out_shape=jax.ShapeDtypeStruct((_N, 64), jnp.float32),
        compiler_params=pltpu.CompilerParams(
            dimension_semantics=("parallel",)),
    )(psums, psums, pcnts, pcnts)


def kernel(node_attr, edge_index, edge_attr, edge_sh, W1, b1, W2, b2):
    out_dtype = jnp.result_type(node_attr.dtype, W1.dtype, W2.dtype,
                                b1.dtype, b2.dtype)
    with jax.enable_x64(False):
        pad = _EP - _E
        src_p = jnp.pad(edge_index[0].astype(jnp.int32), (0, pad))
        dst_p = jnp.pad(edge_index[1].astype(jnp.int32), (0, pad),
                        constant_values=_N)
        ea_p = jnp.pad(edge_attr.astype(jnp.float32), ((0, pad), (0, 0)))
        sh_p = jnp.pad(edge_sh.astype(jnp.float32), ((0, pad), (0, 0)))
        xs = _gather(node_attr.astype(jnp.float32), src_p)
        msgs = _dense(ea_p, xs, sh_p, W1.astype(jnp.float32),
                      b1.astype(jnp.float32).reshape(1, 16),
                      W2.astype(jnp.float32),
                      b2.astype(jnp.float32).reshape(1, 1024))
        zsum = jnp.zeros((_NP, 64), jnp.float32)
        zcnt = jnp.zeros((_NP, _CNTW), jnp.float32)
        ones = jnp.ones((_CH, _CNTW), jnp.float32)
        psums, pcnts = _scatter(msgs, dst_p, zsum, zcnt, ones)
        out = _combine(psums, pcnts)
    return out.astype(out_dtype)


# speedup vs baseline: 198.0298x; 1.0035x over previous
import functools

import numpy as np
import jax
import jax.numpy as jnp
from jax import lax
from jax.experimental import pallas as pl
from jax.experimental.pallas import tpu as pltpu
from jax.experimental.pallas import tpu_sc as plsc

_N = 10000
_E = 160000
_EP = 163840
_NP = 10400
_CH = 128
_NW = 32
_PERW = _EP // _NW
_NCHUNK = _PERW // _CH
_CNTW = 16
_TE = 4096
_TN = 400
_UN = 4

_A0 = float(np.sqrt(1.0 / 32.0))
_A110 = float(np.sqrt(1.0 / 96.0))


def _build_consts():
    e1 = np.zeros((64, 48), np.float32)
    for u in range(16):
        for k in range(3):
            e1[16 + 3 * u + k, 16 * k + u] = 1.0
    r1 = np.zeros((16, 256), np.float32)
    for u in range(16):
        r1[u, u * 16:(u + 1) * 16] = 1.0
    s1 = np.zeros((256, 16), np.float32)
    for j in range(256):
        s1[j, j % 16] = 1.0
    g = np.zeros((48, 48), np.float32)
    for k in range(3):
        for w in range(16):
            g[16 * k + w, 3 * w + k] = 1.0
    return e1, r1, s1, g


_E1, _R1, _S1, _G = _build_consts()


def _dense_body(ea, xs, sh, w1, b1, w2, b2, e1, r1, s1, g, out):
    h = jnp.dot(ea[...], w1[...], preferred_element_type=jnp.float32) + b1[...]
    h = h * jax.nn.sigmoid(h)
    w = jnp.dot(h, w2[...], preferred_element_type=jnp.float32) + b2[...]
    x0 = xs[:, 0:16]
    x1c = jnp.dot(xs[...], e1[...], preferred_element_type=jnp.float32)
    sh0 = sh[:, 0:1]
    v000 = x0 * (sh0 * _A0)
    v110 = (x1c[:, 0:16] * sh[:, 1:2]
            + x1c[:, 16:32] * sh[:, 2:3]
            + x1c[:, 32:48] * sh[:, 3:4]) * _A110
    r = r1[...]
    s = s1[...]
    p0 = (jnp.dot(v000, r, preferred_element_type=jnp.float32) * w[:, 0:256]
          + jnp.dot(v110, r, preferred_element_type=jnp.float32) * w[:, 768:1024])
    out0 = jnp.dot(p0, s, preferred_element_type=jnp.float32)
    t011 = jnp.dot(jnp.dot(x0, r, preferred_element_type=jnp.float32) * w[:, 256:512],
                   s, preferred_element_type=jnp.float32)
    w101 = w[:, 512:768]
    o1 = []
    for k in range(3):
        vk = x1c[:, 16 * k:16 * (k + 1)] * (sh0 * _A0)
        o1k = (jnp.dot(jnp.dot(vk, r, preferred_element_type=jnp.float32) * w101,
                       s, preferred_element_type=jnp.float32)
               + t011 * (sh[:, 1 + k:2 + k] * _A0))
        o1.append(o1k)
    o1cat = jnp.dot(jnp.concatenate(o1, axis=1), g[...],
                    preferred_element_type=jnp.float32)
    out[...] = jnp.concatenate([out0, o1cat], axis=1)


def _dense(ea, xs, sh, w1, b1, w2, b2):
    grid = _EP // _TE
    return pl.pallas_call(
        _dense_body,
        grid=(grid,),
        in_specs=[
            pl.BlockSpec((_TE, 16), lambda i: (i, 0)),
            pl.BlockSpec((_TE, 64), lambda i: (i, 0)),
            pl.BlockSpec((_TE, 4), lambda i: (i, 0)),
            pl.BlockSpec((16, 16), lambda i: (0, 0)),
            pl.BlockSpec((1, 16), lambda i: (0, 0)),
            pl.BlockSpec((16, 1024), lambda i: (0, 0)),
            pl.BlockSpec((1, 1024), lambda i: (0, 0)),
            pl.BlockSpec((64, 48), lambda i: (0, 0)),
            pl.BlockSpec((16, 256), lambda i: (0, 0)),
            pl.BlockSpec((256, 16), lambda i: (0, 0)),
            pl.BlockSpec((48, 48), lambda i: (0, 0)),
        ],
        out_specs=pl.BlockSpec((_TE, 64), lambda i: (i, 0)),
        out_shape=jax.ShapeDtypeStruct((_EP, 64), jnp.float32),
        compiler_params=pltpu.CompilerParams(
            dimension_semantics=("parallel",)),
    )(ea, xs, sh, w1, b1, w2, b2,
      jnp.asarray(_E1), jnp.asarray(_R1), jnp.asarray(_S1), jnp.asarray(_G))


def _gather(nodes, src_p):
    mesh = plsc.VectorSubcoreMesh(core_axis_name="c", subcore_axis_name="s")

    @functools.partial(
        pl.kernel, mesh=mesh,
        out_type=jax.ShapeDtypeStruct((_EP, 64), jnp.float32),
        scratch_types=(
            [pltpu.VMEM((_CH,), jnp.int32) for _ in range(_UN)]
            + [pltpu.VMEM((_CH, 64), jnp.float32) for _ in range(_UN)]
            + [pltpu.SemaphoreType.DMA, pltpu.SemaphoreType.DMA]
        ),
        compiler_params=pltpu.CompilerParams(use_tc_tiling_on_sc=False),
    )
    def gk(nodes_hbm, src_hbm, out_hbm, *bufs):
        idxs = bufs[:_UN]
        rows = bufs[_UN:2 * _UN]
        sem, wsem = bufs[2 * _UN], bufs[2 * _UN + 1]
        wid = lax.axis_index("s") * 2 + lax.axis_index("c")

        def body(i, carry):
            base = wid * np.int32(_PERW) + i * np.int32(_UN * _CH)
            gh = []
            for b in range(_UN):
                off = base + np.int32(b * _CH)
                pltpu.sync_copy(src_hbm.at[pl.ds(off, _CH)], idxs[b])
                gh.append(pltpu.async_copy(nodes_hbm.at[idxs[b]], rows[b], sem))
            wh = []
            for b in range(_UN):
                off = base + np.int32(b * _CH)
                gh[b].wait()
                wh.append(pltpu.async_copy(rows[b], out_hbm.at[pl.ds(off, _CH)],
                                           wsem))
            for b in range(_UN):
                wh[b].wait()
            return carry

        lax.fori_loop(np.int32(0), np.int32(_NCHUNK // _UN), body, np.int32(0))

    return gk(nodes, src_p)


def _scatter(msgs, dst_p, zsum, zcnt, ones):
    mesh = plsc.VectorSubcoreMesh(core_axis_name="c", subcore_axis_name="s")

    @functools.partial(
        pl.kernel, mesh=mesh,
        out_type=(jax.ShapeDtypeStruct((2 * _NP, 64), jnp.float32),
                  jax.ShapeDtypeStruct((2 * _NP, _CNTW), jnp.float32)),
        scratch_types=(
            [pltpu.VMEM((_CH,), jnp.int32) for _ in range(_UN)]
            + [pltpu.VMEM((_CH, 64), jnp.float32) for _ in range(_UN)]
            + [
                pltpu.VMEM((_CH, _CNTW), jnp.float32),
                pltpu.VMEM_SHARED((_NP, 64), jnp.float32),
                pltpu.VMEM_SHARED((_NP, _CNTW), jnp.float32),
                pltpu.SemaphoreType.DMA,
            ]
        ),
        compiler_params=pltpu.CompilerParams(use_tc_tiling_on_sc=False),
    )
    def sk(msgs_hbm, dst_hbm, zs_hbm, zc_hbm, on_hbm,
           sums_out, cnts_out, *bufs):
        idxs = bufs[:_UN]
        rows = bufs[_UN:2 * _UN]
        ones_v, ssum, scnt, sem = bufs[2 * _UN:]
        c = lax.axis_index("c")
        s = lax.axis_index("s")
        wid = s * 2 + c
        rz = _NP // 16
        srz = s * np.int32(rz)
        pltpu.sync_copy(zs_hbm.at[pl.ds(srz, rz)], ssum.at[pl.ds(srz, rz)])
        pltpu.sync_copy(zc_hbm.at[pl.ds(srz, rz)], scnt.at[pl.ds(srz, rz)])
        pltpu.sync_copy(on_hbm, ones_v)
        plsc.subcore_barrier()

        def body(i, carry):
            base = wid * np.int32(_PERW) + i * np.int32(_UN * _CH)
            mh = []
            for b in range(_UN):
                off = base + np.int32(b * _CH)
                pltpu.sync_copy(dst_hbm.at[pl.ds(off, _CH)], idxs[b])
                mh.append(pltpu.async_copy(msgs_hbm.at[pl.ds(off, _CH)],
                                           rows[b], sem))
            for b in range(_UN):
                mh[b].wait()
                pltpu.sync_copy(rows[b], ssum.at[idxs[b]], add=True)
                pltpu.sync_copy(ones_v, scnt.at[idxs[b]], add=True)
            return carry

        lax.fori_loop(np.int32(0), np.int32(_NCHUNK // _UN), body, np.int32(0))
        plsc.subcore_barrier()
        obase = c * np.int32(_NP) + srz
        pltpu.sync_copy(ssum.at[pl.ds(srz, rz)], sums_out.at[pl.ds(obase, rz)])
        pltpu.sync_copy(scnt.at[pl.ds(srz, rz)], cnts_out.at[pl.ds(obase, rz)])

    return sk(msgs, dst_p, zsum, zcnt, ones)


def _combine_body(ps_a, ps_b, pc_a, pc_b, out):
    cnt = pc_a[:, 0:1] + pc_b[:, 0:1]
    out[...] = (ps_a[...] + ps_b[...]) / jnp.maximum(cnt, 1.0)


def _combine(psums, pcnts):
    nb = _NP // _TN
    return pl.pallas_call(
        _combine_body,
        grid=(_N // _TN,),
        in_specs=[
            pl.BlockSpec((_TN, 64), lambda i: (i, 0)),
            pl.BlockSpec((_TN, 64), lambda i: (i + nb, 0)),
            pl.BlockSpec((_TN, _CNTW), lambda i: (i, 0)),
            pl.BlockSpec((_TN, _CNTW), lambda i: (i + nb, 0)),
        ],
        out_specs=pl.BlockSpec((_TN, 64), lambda i: (i, 0)),
        out_shape=jax.ShapeDtypeStruct((_N, 64), jnp.float32),
        compiler_params=pltpu.CompilerParams(
            dimension_semantics=("parallel",)),
    )(psums, psums, pcnts, pcnts)


def kernel(node_attr, edge_index, edge_attr, edge_sh, W1, b1, W2, b2):
    out_dtype = jnp.result_type(node_attr.dtype, W1.dtype, W2.dtype,
                                b1.dtype, b2.dtype)
    with jax.enable_x64(False):
        pad = _EP - _E
        src_p = jnp.pad(edge_index[0].astype(jnp.int32), (0, pad))
        dst_p = jnp.pad(edge_index[1].astype(jnp.int32), (0, pad),
                        constant_values=_N)
        ea_p = jnp.pad(edge_attr.astype(jnp.float32), ((0, pad), (0, 0)))
        sh_p = jnp.pad(edge_sh.astype(jnp.float32), ((0, pad), (0, 0)))
        xs = _gather(node_attr.astype(jnp.float32), src_p)
        msgs = _dense(ea_p, xs, sh_p, W1.astype(jnp.float32),
                      b1.astype(jnp.float32).reshape(1, 16),
                      W2.astype(jnp.float32),
                      b2.astype(jnp.float32).reshape(1, 1024))
        zsum = jnp.zeros((_NP, 64), jnp.float32)
        zcnt = jnp.zeros((_NP, _CNTW), jnp.float32)
        ones = jnp.ones((_CH, _CNTW), jnp.float32)
        psums, pcnts = _scatter(msgs, dst_p, zsum, zcnt, ones)
        out = _combine(psums, pcnts)
    return out.astype(out_dtype)


# speedup vs baseline: 198.2755x; 1.0012x over previous
import functools

import numpy as np
import jax
import jax.numpy as jnp
from jax import lax
from jax.experimental import pallas as pl
from jax.experimental.pallas import tpu as pltpu
from jax.experimental.pallas import tpu_sc as plsc

_N = 10000
_E = 160000
_EP = 163840
_NP = 10400
_CH = 128
_NW = 32
_PERW = _EP // _NW
_NCHUNK = _PERW // _CH
_CNTW = 16
_TE = 4096
_TN = 400
_UN = 8

_A0 = float(np.sqrt(1.0 / 32.0))
_A110 = float(np.sqrt(1.0 / 96.0))


def _build_consts():
    e1 = np.zeros((64, 48), np.float32)
    for u in range(16):
        for k in range(3):
            e1[16 + 3 * u + k, 16 * k + u] = 1.0
    r1 = np.zeros((16, 256), np.float32)
    for u in range(16):
        r1[u, u * 16:(u + 1) * 16] = 1.0
    s1 = np.zeros((256, 16), np.float32)
    for j in range(256):
        s1[j, j % 16] = 1.0
    g = np.zeros((48, 48), np.float32)
    for k in range(3):
        for w in range(16):
            g[16 * k + w, 3 * w + k] = 1.0
    return e1, r1, s1, g


_E1, _R1, _S1, _G = _build_consts()


def _dense_body(ea, xs, sh, w1, b1, w2, b2, e1, r1, s1, g, out):
    h = jnp.dot(ea[...], w1[...], preferred_element_type=jnp.float32) + b1[...]
    h = h * jax.nn.sigmoid(h)
    w = jnp.dot(h, w2[...], preferred_element_type=jnp.float32) + b2[...]
    x0 = xs[:, 0:16]
    x1c = jnp.dot(xs[...], e1[...], preferred_element_type=jnp.float32)
    sh0 = sh[:, 0:1]
    v000 = x0 * (sh0 * _A0)
    v110 = (x1c[:, 0:16] * sh[:, 1:2]
            + x1c[:, 16:32] * sh[:, 2:3]
            + x1c[:, 32:48] * sh[:, 3:4]) * _A110
    r = r1[...]
    s = s1[...]
    p0 = (jnp.dot(v000, r, preferred_element_type=jnp.float32) * w[:, 0:256]
          + jnp.dot(v110, r, preferred_element_type=jnp.float32) * w[:, 768:1024])
    out0 = jnp.dot(p0, s, preferred_element_type=jnp.float32)
    t011 = jnp.dot(jnp.dot(x0, r, preferred_element_type=jnp.float32) * w[:, 256:512],
                   s, preferred_element_type=jnp.float32)
    w101 = w[:, 512:768]
    o1 = []
    for k in range(3):
        vk = x1c[:, 16 * k:16 * (k + 1)] * (sh0 * _A0)
        o1k = (jnp.dot(jnp.dot(vk, r, preferred_element_type=jnp.float32) * w101,
                       s, preferred_element_type=jnp.float32)
               + t011 * (sh[:, 1 + k:2 + k] * _A0))
        o1.append(o1k)
    o1cat = jnp.dot(jnp.concatenate(o1, axis=1), g[...],
                    preferred_element_type=jnp.float32)
    out[...] = jnp.concatenate([out0, o1cat], axis=1)


def _dense(ea, xs, sh, w1, b1, w2, b2):
    grid = _EP // _TE
    return pl.pallas_call(
        _dense_body,
        grid=(grid,),
        in_specs=[
            pl.BlockSpec((_TE, 16), lambda i: (i, 0)),
            pl.BlockSpec((_TE, 64), lambda i: (i, 0)),
            pl.BlockSpec((_TE, 4), lambda i: (i, 0)),
            pl.BlockSpec((16, 16), lambda i: (0, 0)),
            pl.BlockSpec((1, 16), lambda i: (0, 0)),
            pl.BlockSpec((16, 1024), lambda i: (0, 0)),
            pl.BlockSpec((1, 1024), lambda i: (0, 0)),
            pl.BlockSpec((64, 48), lambda i: (0, 0)),
            pl.BlockSpec((16, 256), lambda i: (0, 0)),
            pl.BlockSpec((256, 16), lambda i: (0, 0)),
            pl.BlockSpec((48, 48), lambda i: (0, 0)),
        ],
        out_specs=pl.BlockSpec((_TE, 64), lambda i: (i, 0)),
        out_shape=jax.ShapeDtypeStruct((_EP, 64), jnp.float32),
        compiler_params=pltpu.CompilerParams(
            dimension_semantics=("parallel",)),
    )(ea, xs, sh, w1, b1, w2, b2,
      jnp.asarray(_E1), jnp.asarray(_R1), jnp.asarray(_S1), jnp.asarray(_G))


def _gather(nodes, src_p):
    mesh = plsc.VectorSubcoreMesh(core_axis_name="c", subcore_axis_name="s")

    @functools.partial(
        pl.kernel, mesh=mesh,
        out_type=jax.ShapeDtypeStruct((_EP, 64), jnp.float32),
        scratch_types=(
            [pltpu.VMEM((_CH,), jnp.int32) for _ in range(_UN)]
            + [pltpu.VMEM((_CH, 64), jnp.float32) for _ in range(_UN)]
            + [pltpu.SemaphoreType.DMA, pltpu.SemaphoreType.DMA]
        ),
        compiler_params=pltpu.CompilerParams(use_tc_tiling_on_sc=False),
    )
    def gk(nodes_hbm, src_hbm, out_hbm, *bufs):
        idxs = bufs[:_UN]
        rows = bufs[_UN:2 * _UN]
        sem, wsem = bufs[2 * _UN], bufs[2 * _UN + 1]
        wid = lax.axis_index("s") * 2 + lax.axis_index("c")

        def body(i, carry):
            base = wid * np.int32(_PERW) + i * np.int32(_UN * _CH)
            gh = []
            for b in range(_UN):
                off = base + np.int32(b * _CH)
                pltpu.sync_copy(src_hbm.at[pl.ds(off, _CH)], idxs[b])
                gh.append(pltpu.async_copy(nodes_hbm.at[idxs[b]], rows[b], sem))
            wh = []
            for b in range(_UN):
                off = base + np.int32(b * _CH)
                gh[b].wait()
                wh.append(pltpu.async_copy(rows[b], out_hbm.at[pl.ds(off, _CH)],
                                           wsem))
            for b in range(_UN):
                wh[b].wait()
            return carry

        lax.fori_loop(np.int32(0), np.int32(_NCHUNK // _UN), body, np.int32(0))

    return gk(nodes, src_p)


def _scatter(msgs, dst_p, zsum, zcnt, ones):
    mesh = plsc.VectorSubcoreMesh(core_axis_name="c", subcore_axis_name="s")

    @functools.partial(
        pl.kernel, mesh=mesh,
        out_type=(jax.ShapeDtypeStruct((2 * _NP, 64), jnp.float32),
                  jax.ShapeDtypeStruct((2 * _NP, _CNTW), jnp.float32)),
        scratch_types=(
            [pltpu.VMEM((_CH,), jnp.int32) for _ in range(_UN)]
            + [pltpu.VMEM((_CH, 64), jnp.float32) for _ in range(_UN)]
            + [
                pltpu.VMEM((_CH, _CNTW), jnp.float32),
                pltpu.VMEM_SHARED((_NP, 64), jnp.float32),
                pltpu.VMEM_SHARED((_NP, _CNTW), jnp.float32),
                pltpu.SemaphoreType.DMA,
            ]
        ),
        compiler_params=pltpu.CompilerParams(use_tc_tiling_on_sc=False),
    )
    def sk(msgs_hbm, dst_hbm, zs_hbm, zc_hbm, on_hbm,
           sums_out, cnts_out, *bufs):
        idxs = bufs[:_UN]
        rows = bufs[_UN:2 * _UN]
        ones_v, ssum, scnt, sem = bufs[2 * _UN:]
        c = lax.axis_index("c")
        s = lax.axis_index("s")
        wid = s * 2 + c
        rz = _NP // 16
        srz = s * np.int32(rz)
        pltpu.sync_copy(zs_hbm.at[pl.ds(srz, rz)], ssum.at[pl.ds(srz, rz)])
        pltpu.sync_copy(zc_hbm.at[pl.ds(srz, rz)], scnt.at[pl.ds(srz, rz)])
        pltpu.sync_copy(on_hbm, ones_v)
        plsc.subcore_barrier()

        def body(i, carry):
            base = wid * np.int32(_PERW) + i * np.int32(_UN * _CH)
            mh = []
            for b in range(_UN):
                off = base + np.int32(b * _CH)
                pltpu.sync_copy(dst_hbm.at[pl.ds(off, _CH)], idxs[b])
                mh.append(pltpu.async_copy(msgs_hbm.at[pl.ds(off, _CH)],
                                           rows[b], sem))
            for b in range(_UN):
                mh[b].wait()
                pltpu.sync_copy(rows[b], ssum.at[idxs[b]], add=True)
                pltpu.sync_copy(ones_v, scnt.at[idxs[b]], add=True)
            return carry

        lax.fori_loop(np.int32(0), np.int32(_NCHUNK // _UN), body, np.int32(0))
        plsc.subcore_barrier()
        obase = c * np.int32(_NP) + srz
        pltpu.sync_copy(ssum.at[pl.ds(srz, rz)], sums_out.at[pl.ds(obase, rz)])
        pltpu.sync_copy(scnt.at[pl.ds(srz, rz)], cnts_out.at[pl.ds(obase, rz)])

    return sk(msgs, dst_p, zsum, zcnt, ones)


def _combine_body(ps_a, ps_b, pc_a, pc_b, out):
    cnt = pc_a[:, 0:1] + pc_b[:, 0:1]
    out[...] = (ps_a[...] + ps_b[...]) / jnp.maximum(cnt, 1.0)


def _combine(psums, pcnts):
    nb = _NP // _TN
    return pl.pallas_call(
        _combine_body,
        grid=(_N // _TN,),
        in_specs=[
            pl.BlockSpec((_TN, 64), lambda i: (i, 0)),
            pl.BlockSpec((_TN, 64), lambda i: (i + nb, 0)),
            pl.BlockSpec((_TN, _CNTW), lambda i: (i, 0)),
            pl.BlockSpec((_TN, _CNTW), lambda i: (i + nb, 0)),
        ],
        out_specs=pl.BlockSpec((_TN, 64), lambda i: (i, 0)),
        out_shape=jax.ShapeDtypeStruct((_N, 64), jnp.float32),
        compiler_params=pltpu.CompilerParams(
            dimension_semantics=("parallel",)),
    )(psums, psums, pcnts, pcnts)


def kernel(node_attr, edge_index, edge_attr, edge_sh, W1, b1, W2, b2):
    out_dtype = jnp.result_type(node_attr.dtype, W1.dtype, W2.dtype,
                                b1.dtype, b2.dtype)
    with jax.enable_x64(False):
        pad = _EP - _E
        src_p = jnp.pad(edge_index[0].astype(jnp.int32), (0, pad))
        dst_p = jnp.pad(edge_index[1].astype(jnp.int32), (0, pad),
                        constant_values=_N)
        ea_p = jnp.pad(edge_attr.astype(jnp.float32), ((0, pad), (0, 0)))
        sh_p = jnp.pad(edge_sh.astype(jnp.float32), ((0, pad), (0, 0)))
        xs = _gather(node_attr.astype(jnp.float32), src_p)
        msgs = _dense(ea_p, xs, sh_p, W1.astype(jnp.float32),
                      b1.astype(jnp.float32).reshape(1, 16),
                      W2.astype(jnp.float32),
                      b2.astype(jnp.float32).reshape(1, 1024))
        zsum = jnp.zeros((_NP, 64), jnp.float32)
        zcnt = jnp.zeros((_NP, _CNTW), jnp.float32)
        ones = jnp.ones((_CH, _CNTW), jnp.float32)
        psums, pcnts = _scatter(msgs, dst_p, zsum, zcnt, ones)
        out = _combine(psums, pcnts)
    return out.astype(out_dtype)
